# baseline (speedup 1.0000x reference)
import dataclasses

import jax
import jax.numpy as jnp
from jax.experimental import pallas as pl
from jax.experimental.pallas import tpu as pltpu
from jax.experimental.pallas import tpu_sc as plsc

N = 320000
D = 128
LANES = 16
ROWS_PER_BLK = 128
NGROUPS = ROWS_PER_BLK // LANES


def _compiler_params():
    cp = pltpu.CompilerParams()
    if "needs_layout_passes" in pltpu.CompilerParams.__dataclass_fields__:
        cp = dataclasses.replace(cp, needs_layout_passes=False)
    return cp


def _sc_mask_rows(score_t, y):
    mesh = plsc.VectorSubcoreMesh(core_axis_name="core", subcore_axis_name="subcore")

    @pl.kernel(
        out_type=jax.ShapeDtypeStruct((N, D), jnp.float32),
        mesh=mesh,
        scratch_types=[pltpu.VMEM((ROWS_PER_BLK,), jnp.float32)],
        compiler_params=_compiler_params(),
    )
    def sc_kernel(score_t_hbm, y_hbm, o_hbm, mask_ref):
        def body(st_v, y_v, o_v):
            @pl.loop(0, NGROUPS)
            def _(g):
                base = g * LANES
                c0 = st_v[0, pl.ds(base, LANES)]
                c1 = st_v[1, pl.ds(base, LANES)]
                c2 = st_v[2, pl.ds(base, LANES)]
                c3 = st_v[3, pl.ds(base, LANES)]
                ss = c0 * c0 + c1 * c1 + c2 * c2 + c3 * c3
                mask_ref[pl.ds(base, LANES)] = jnp.where(
                    ss >= 4.0, jnp.float32(1.0), jnp.float32(0.0)
                )

            @pl.loop(0, ROWS_PER_BLK)
            def _(row):
                m = plsc.load_gather(
                    mask_ref, [jnp.full((LANES,), row, jnp.int32)]
                )

                @pl.loop(0, D, step=LANES)
                def _(c):
                    o_v[row, pl.ds(c, LANES)] = y_v[row, pl.ds(c, LANES)] * m

        pltpu.emit_pipeline(
            body,
            grid=(N // ROWS_PER_BLK,),
            in_specs=[
                pl.BlockSpec((4, ROWS_PER_BLK), lambda i: (0, i)),
                pl.BlockSpec((ROWS_PER_BLK, D), lambda i: (i, 0)),
            ],
            out_specs=[
                pl.BlockSpec((ROWS_PER_BLK, D), lambda i: (i, 0)),
            ],
            core_axis_name=("core", "subcore"),
            dimension_semantics=(pltpu.PARALLEL,),
        )(score_t_hbm, y_hbm, o_hbm)

    return sc_kernel(score_t, y)


def kernel(edge_index, score, y):
    del edge_index
    score_t = score.T
    return _sc_mask_rows(score_t, y)
